# baseline (speedup 1.0000x reference)
import functools

import jax
import jax.numpy as jnp
from jax import lax
from jax.experimental import pallas as pl
from jax.experimental.pallas import tpu as pltpu
from jax.experimental.pallas import tpu_sc as plsc

_N_IDX = 153
_N_PAD = 160
_H = 512
_W = 512
_BLK = 12
_MASK_ROWS = 8


def _sc_mask_body(idxh_hbm, idxw_hbm, out_hbm, idx_v, mask_v):
    cid = lax.axis_index("c")
    sid = lax.axis_index("s")

    def build(idx_hbm, row):
        pltpu.sync_copy(idx_hbm, idx_v)
        ones = jnp.ones((16,), jnp.float32)
        for j in range(_H // 16):
            mask_v[pl.ds(j * 16, 16)] = ones
        zeros = jnp.zeros((16,), jnp.float32)
        for j in range(_N_PAD // 16):
            plsc.store_scatter(mask_v, [idx_v[pl.ds(j * 16, 16)]], zeros)
        pltpu.sync_copy(mask_v, out_hbm.at[row])

    @pl.when(jnp.logical_and(sid == 0, cid == 0))
    def _():
        build(idxh_hbm, 0)

    @pl.when(jnp.logical_and(sid == 0, cid == 1))
    def _():
        build(idxw_hbm, 1)


def _sc_masks(ih, iw):
    return pl.kernel(
        _sc_mask_body,
        out_type=jax.ShapeDtypeStruct((_MASK_ROWS, _H), jnp.float32),
        mesh=plsc.VectorSubcoreMesh(core_axis_name="c", subcore_axis_name="s"),
        scratch_types=[
            pltpu.VMEM((_N_PAD,), jnp.int32),
            pltpu.VMEM((_H,), jnp.float32),
        ],
        compiler_params=pltpu.CompilerParams(needs_layout_passes=False),
    )(ih, iw)


def _mask_mul_kernel(m_ref, x_ref, o_ref):
    kh_row = m_ref[0:1, :]
    kw_row = m_ref[1:2, :]
    ones11 = jnp.ones((1, 1), jnp.float32)
    kh_col = lax.dot_general(
        kh_row, ones11, (((0,), (0,)), ((), ())),
        preferred_element_type=jnp.float32)
    mask2d = kh_col * kw_row
    o_ref[...] = x_ref[...] * mask2d[None, :, :]


@jax.jit
def kernel(noised, idx_h, idx_w):
    b, c, h, w = noised.shape
    x = noised.reshape(b * c, h, w)
    n = b * c

    def padded(idx):
        idx = idx.astype(jnp.int32)
        return jnp.concatenate([idx, jnp.broadcast_to(idx[0], (_N_PAD - _N_IDX,))])

    masks = _sc_masks(padded(idx_h), padded(idx_w))

    out = pl.pallas_call(
        _mask_mul_kernel,
        grid=(n // _BLK,),
        in_specs=[
            pl.BlockSpec((_MASK_ROWS, _H), lambda i: (0, 0)),
            pl.BlockSpec((_BLK, h, w), lambda i: (i, 0, 0)),
        ],
        out_specs=pl.BlockSpec((_BLK, h, w), lambda i: (i, 0, 0)),
        out_shape=jax.ShapeDtypeStruct((n, h, w), noised.dtype),
        compiler_params=pltpu.CompilerParams(
            dimension_semantics=("arbitrary",),
        ),
    )(masks, x)
    return out.reshape(b, c, h, w)
